# Initial kernel scaffold; baseline (speedup 1.0000x reference)
import jax
import jax.numpy as jnp
from jax.experimental import pallas as pl


def kernel(x, w_conv1, b_conv1, w_conv2, b_conv2, w_fc1, b_fc1, gamma, beta, w_fc2, b_fc2):
    raise NotImplementedError("write your pallas kernel here")


# speedup vs baseline: 1.5938x; 1.5938x over previous
import functools

import jax
import jax.numpy as jnp
from jax.experimental import pallas as pl
from jax.experimental.pallas import tpu as pltpu

_BF = jnp.bfloat16
_F32 = jnp.float32


def _feature_fwd(x_ref, m_ref, w1_ref, b1_ref, w2_ref, b2_ref, v1_ref, bf1_ref,
                 z_ref, slab, pp, f, *, bc, H, W, Cin, C1, C2, pad):
    HW = H * W
    N = bc * HW

    slab[:, 0:pad] = jnp.zeros((slab.shape[0], pad), _BF)
    slab[:, pad + N:pad + N + pad] = jnp.zeros((slab.shape[0], pad), _BF)

    def im2col(rows, masked):
        for k in range(9):
            dy, dx = k // 3 - 1, k % 3 - 1
            delta = dy * W + dx
            src = slab[0:rows, pad + delta:pad + delta + N]
            if k == 4:
                pp[k * rows:(k + 1) * rows, :] = src
            else:
                mrow = k if k < 4 else k - 1
                pp[k * rows:(k + 1) * rows, :] = (src * m_ref[mrow:mrow + 1, :]
                                                  if masked else src)

    slab[0:Cin, pad:pad + N] = x_ref[...]
    im2col(Cin, True)
    y1 = jnp.dot(w1_ref[...], pp[0:9 * Cin, :], preferred_element_type=_F32)
    y1 = jnp.maximum(y1 + b1_ref[...], 0.0).astype(_BF)

    slab[0:C1, pad:pad + N] = y1
    im2col(C1, True)
    y2 = jnp.dot(w2_ref[...], pp[0:9 * C1, :], preferred_element_type=_F32)
    slab[0:C2, pad:pad + N] = (y2 + b2_ref[...]).astype(_BF)

    t = jnp.maximum(
        jnp.maximum(slab[0:C2, pad:pad + N], slab[0:C2, pad + 1:pad + 1 + N]),
        jnp.maximum(slab[0:C2, pad + W:pad + W + N],
                    slab[0:C2, pad + W + 1:pad + W + 1 + N]))

    f[...] = jnp.transpose(t.reshape(C2, bc, HW), (1, 0, 2)).reshape(bc, C2 * HW)

    z_ref[...] = (jnp.dot(f[...], v1_ref[...], preferred_element_type=_F32)
                  + bf1_ref[...])


def _head_fwd(z_ref, g_ref, be_ref, w3_ref, b3_ref, o_ref):
    z = z_ref[...]
    mu = jnp.mean(z, axis=0, keepdims=True)
    d = z - mu
    var = jnp.mean(d * d, axis=0, keepdims=True)
    h = g_ref[...] * (d * jax.lax.rsqrt(var + 1e-5)) + be_ref[...]
    h = jnp.maximum(h, 0.0).astype(_BF)
    o_ref[...] = (jnp.dot(h, w3_ref[...], preferred_element_type=_F32)
                  + b3_ref[...])


def _pack_conv(w):
    o, c, kh, kw = w.shape
    return jnp.transpose(w, (0, 2, 3, 1)).reshape(o, kh * kw * c).astype(_BF)


def _fold_fc1(w_fc1, C2, H, W):
    hid = w_fc1.shape[0]
    Hp, Wp = H // 2, W // 2
    v6 = w_fc1.T.reshape(C2, Hp, 1, Wp, 1, hid)
    v6 = jnp.pad(v6, ((0, 0), (0, 0), (0, 1), (0, 0), (0, 1), (0, 0)))
    return v6.reshape(C2 * H * W, hid).astype(_BF)


def _edge_masks(H, W, bc):
    s = jnp.arange(H * W)
    hh, ww = s // W, s % W
    rows = []
    for k in range(9):
        if k == 4:
            continue
        dy, dx = k // 3 - 1, k % 3 - 1
        ok = (hh + dy >= 0) & (hh + dy < H) & (ww + dx >= 0) & (ww + dx < W)
        rows.append(ok)
    m = jnp.stack(rows, axis=0).astype(_BF)
    return jnp.tile(m, (1, bc))


@jax.jit
def _forward(x, w1, b1, w2, b2, wf1, bf1, gamma, beta, wf2, bf2):
    B, Cin, H, W = x.shape
    C1, C2 = w1.shape[0], w2.shape[0]
    hid = wf1.shape[0]
    ncls = wf2.shape[0]
    HW = H * W

    bc = 16 if (B % 16 == 0 and (16 * HW) % 128 == 0) else 8
    nchunk = B // bc
    pad = 128

    x2 = jnp.transpose(x, (1, 0, 2, 3)).reshape(Cin, B * HW).astype(_BF)
    masks = _edge_masks(H, W, bc)
    w1p = _pack_conv(w1)
    w2p = _pack_conv(w2)
    b1c = b1.reshape(C1, 1).astype(_F32)
    b2c = b2.reshape(C2, 1).astype(_F32)
    v1 = _fold_fc1(wf1, C2, H, W)
    bf1r = bf1.reshape(1, hid).astype(_F32)

    feat = functools.partial(_feature_fwd, bc=bc, H=H, W=W, Cin=Cin,
                             C1=C1, C2=C2, pad=pad)
    z = pl.pallas_call(
        feat,
        out_shape=jax.ShapeDtypeStruct((nchunk, bc, hid), _F32),
        grid_spec=pltpu.PrefetchScalarGridSpec(
            num_scalar_prefetch=0,
            grid=(nchunk,),
            in_specs=[
                pl.BlockSpec((Cin, bc * HW), lambda i: (0, i)),
                pl.BlockSpec((8, bc * HW), lambda i: (0, 0)),
                pl.BlockSpec((C1, 9 * Cin), lambda i: (0, 0)),
                pl.BlockSpec((C1, 1), lambda i: (0, 0)),
                pl.BlockSpec((C2, 9 * C1), lambda i: (0, 0)),
                pl.BlockSpec((C2, 1), lambda i: (0, 0)),
                pl.BlockSpec((C2 * HW, hid), lambda i: (0, 0)),
                pl.BlockSpec((1, hid), lambda i: (0, 0)),
            ],
            out_specs=pl.BlockSpec((None, bc, hid), lambda i: (i, 0, 0)),
            scratch_shapes=[
                pltpu.VMEM((max(Cin, C1, C2), bc * HW + 2 * pad), _BF),
                pltpu.VMEM((9 * C1, bc * HW), _BF),
                pltpu.VMEM((bc, C2 * HW), _BF),
            ]),
        compiler_params=pltpu.CompilerParams(
            dimension_semantics=("parallel",)),
    )(x2, masks, w1p, b1c, w2p, b2c, v1, bf1r)

    return pl.pallas_call(
        _head_fwd,
        out_shape=jax.ShapeDtypeStruct((B, ncls), _F32),
    )(z.reshape(B, hid), gamma.reshape(1, hid).astype(_F32),
      beta.reshape(1, hid).astype(_F32), wf2.T.astype(_BF),
      bf2.reshape(1, ncls).astype(_F32))


def kernel(x, w_conv1, b_conv1, w_conv2, b_conv2, w_fc1, b_fc1, gamma, beta,
           w_fc2, b_fc2):
    return _forward(x, w_conv1, b_conv1, w_conv2, b_conv2, w_fc1, b_fc1,
                    gamma, beta, w_fc2, b_fc2)
